# Initial kernel scaffold; baseline (speedup 1.0000x reference)
import jax
import jax.numpy as jnp
from jax.experimental import pallas as pl


def kernel(dist, e_type, senders, receivers, W1, b1, W2, b2, Wh, bh, Wg, bg, Y_emb, X_emb):
    raise NotImplementedError("write your pallas kernel here")


# speedup vs baseline: 3.6730x; 3.6730x over previous
import functools

import jax
import jax.numpy as jnp
from jax import lax
from jax.experimental import pallas as pl
from jax.experimental.pallas import tpu as pltpu
from jax.experimental.pallas import tpu_sc as plsc

N_NUC = 512
N_ELEC = 4096
N_NODES = N_NUC + N_ELEC
E = 73728
DIST = 16
KD = 128
ED = 64
L = 2

LN2 = 0.6931471805599453

NC = 2
NS = 16
HKD = KD // NC
C = 128
EPT = E // NS
CPT = EPT // C
CB = C * HKD * 4
ZROWS = 3 * N_ELEC
DUMP = ZROWS
ZPAD = ZROWS + 8
ROWS_PER_TILE = ZROWS // NS


def _ssp(x):
    return jnp.maximum(x, 0.0) + jnp.log1p(jnp.exp(-jnp.abs(x))) - LN2


def _edge_w_body(dist_ref, et_ref, w1_ref, b1_ref, w2_ref, b2_ref, we_ref):
    x = dist_ref[...]
    h = jnp.dot(x, w1_ref[...], preferred_element_type=jnp.float32)
    h = _ssp(h + b1_ref[...])
    et = et_ref[...]
    blk = x.shape[0]
    m_same = (et == 3).astype(jnp.float32)
    m_anti = (et == 4).astype(jnp.float32)
    m_nuc = (et == 1).astype(jnp.float32)
    mask48 = jnp.concatenate(
        [jnp.broadcast_to(m_same, (blk, DIST)),
         jnp.broadcast_to(m_anti, (blk, DIST)),
         jnp.broadcast_to(m_nuc, (blk, DIST))], axis=1)
    we = jnp.dot(h * mask48, w2_ref[...], preferred_element_type=jnp.float32)
    we = we + (m_same * b2_ref[0:1, :] + m_anti * b2_ref[1:2, :]
               + m_nuc * b2_ref[2:3, :])
    we_ref[0] = we[:, :HKD]
    we_ref[1] = we[:, HKD:]


def _edge_we(dist, et2, w1c, b1c, w2c, b2c):
    blk = 2048
    grid = (E // blk,)
    return pl.pallas_call(
        _edge_w_body,
        grid=grid,
        in_specs=[
            pl.BlockSpec((blk, DIST), lambda i: (i, 0)),
            pl.BlockSpec((blk, 1), lambda i: (i, 0)),
            pl.BlockSpec((DIST, 3 * DIST), lambda i: (0, 0)),
            pl.BlockSpec((1, 3 * DIST), lambda i: (0, 0)),
            pl.BlockSpec((3 * DIST, KD), lambda i: (0, 0)),
            pl.BlockSpec((3, KD), lambda i: (0, 0)),
        ],
        out_specs=pl.BlockSpec((NC, blk, HKD), lambda i: (0, i, 0)),
        out_shape=jax.ShapeDtypeStruct((NC, E, HKD), jnp.float32),
    )(dist, et2, w1c, b1c, w2c, b2c)


def _sidx_body(et_ref, rc_ref, out_ref):
    et = et_ref[...]
    rc = rc_ref[...]
    bucket = jnp.where(et == 3, 0, jnp.where(et == 4, 1, jnp.where(et == 1, 2, 3)))
    live = (bucket < 3) & (rc >= N_NUC)
    out_ref[...] = jnp.where(live, bucket * N_ELEC + rc - N_NUC, DUMP).astype(jnp.int32)


def _make_sidx(et2, rc2):
    blk = 4096
    return pl.pallas_call(
        _sidx_body,
        grid=(E // blk,),
        in_specs=[pl.BlockSpec((blk, 1), lambda i: (i, 0)),
                  pl.BlockSpec((blk, 1), lambda i: (i, 0))],
        out_specs=pl.BlockSpec((blk, 1), lambda i: (i, 0)),
        out_shape=jax.ShapeDtypeStruct((E, 1), jnp.int32),
    )(et2, rc2)


def _embed_body(elec_ref, wh_ref, bh_ref, y_ref, outa_ref, outb_ref):
    ex = jnp.dot(elec_ref[...], wh_ref[...],
                 preferred_element_type=jnp.float32) + bh_ref[...]
    y = y_ref[...]
    outa_ref[:N_NUC, :] = y[:, :HKD]
    outa_ref[N_NUC:, :] = ex[:, :HKD]
    outb_ref[:N_NUC, :] = y[:, HKD:]
    outb_ref[N_NUC:, :] = ex[:, HKD:]


def _embed_hx(elec, wh, bh_row, y_emb):
    return pl.pallas_call(
        _embed_body,
        out_shape=[jax.ShapeDtypeStruct((N_NODES, HKD), jnp.float32),
                   jax.ShapeDtypeStruct((N_NODES, HKD), jnp.float32)],
    )(elec, wh, bh_row, y_emb)


def _update_body(z_ref, wg_ref, bg_ref, elec_ref, out_ref):
    acc = elec_ref[...] + bg_ref[...]
    for c in range(NC):
        for t in range(3):
            zt = z_ref[(c * 3 + t) * N_ELEC:(c * 3 + t + 1) * N_ELEC, :]
            acc = acc + jnp.dot(zt, wg_ref[t, c * HKD:(c + 1) * HKD, :],
                                preferred_element_type=jnp.float32)
    out_ref[...] = acc


def _update(z, wg, bg_row, elec):
    return pl.pallas_call(
        _update_body,
        out_shape=jax.ShapeDtypeStruct((N_ELEC, ED), jnp.float32),
    )(z, wg, bg_row, elec)


def _sc_body(hxa, hxb, we2, snd3, sidx3, out,
             z_sh, snd_loc, sidx_loc, rows_b, we_b,
             gs0, gs1, gs2, ws0, ws1, ws2, ss0, ss1, ss2):
    c = lax.axis_index("c")
    s = lax.axis_index("s")
    gsem = (gs0, gs1, gs2)
    wsem = (ws0, ws1, ws2)
    ssem = (ss0, ss1, ss2)

    pltpu.sync_copy(snd3.at[s], snd_loc)
    pltpu.sync_copy(sidx3.at[s], sidx_loc)

    zv = jnp.zeros((16,), jnp.float32)

    def _zrow(r, _):
        for j in range(HKD // 16):
            we_b[0, r, pl.ds(j * 16, 16)] = zv
        return 0

    lax.fori_loop(0, C, _zrow, 0)
    for k in range(ROWS_PER_TILE // C):
        pltpu.sync_copy(we_b.at[0], z_sh.at[pl.ds(s * ROWS_PER_TILE + k * C, C), :])
    plsc.subcore_barrier()

    ebase = s * EPT

    def _drain(sem, buf):
        pltpu.make_async_copy(we2.at[c, pl.ds(0, C), :], buf, sem).wait()

    def _issue(k, b):
        base = ebase + k * C
        pltpu.async_copy(we2.at[c, pl.ds(base, C), :], we_b.at[b], wsem[b])

        @pl.when(c == 0)
        def _():
            pltpu.async_copy(hxa.at[snd_loc.at[k]], rows_b.at[b], gsem[b])

        @pl.when(c == 1)
        def _():
            pltpu.async_copy(hxb.at[snd_loc.at[k]], rows_b.at[b], gsem[b])

    _issue(0, 0)
    _issue(1, 1)

    def _outer(ko, _):
        for j in range(3):
            k = 3 * ko + j
            b = j
            b2 = (j + 2) % 3
            _drain(wsem[b], we_b.at[b])
            _drain(gsem[b], rows_b.at[b])

            def _mrow(r, _):
                for q in range(HKD // 16):
                    sl = pl.ds(q * 16, 16)
                    rows_b[b, r, sl] = rows_b[b, r, sl] * we_b[b, r, sl]
                return 0

            lax.fori_loop(0, C, _mrow, 0)
            pltpu.async_copy(rows_b.at[b], z_sh.at[sidx_loc.at[k]], ssem[b],
                             add=True)

            @pl.when(k >= 1)
            def _():
                _drain(ssem[b2], rows_b.at[b2])

            @pl.when(k + 2 < CPT)
            def _():
                _issue(k + 2, b2)
        return 0

    lax.fori_loop(0, CPT // 3, _outer, 0)
    _drain(ssem[(CPT - 1) % 3], rows_b.at[(CPT - 1) % 3])
    plsc.subcore_barrier()

    for k in range(ROWS_PER_TILE // C):
        row = s * ROWS_PER_TILE + k * C
        pltpu.sync_copy(z_sh.at[pl.ds(row, C), :], rows_b.at[0])
        pltpu.sync_copy(rows_b.at[0], out.at[pl.ds(c * ZROWS + row, C), :])


def _sc_message(hxa, hxb, we2, snd3, sidx3):
    mesh = plsc.VectorSubcoreMesh(core_axis_name="c", subcore_axis_name="s",
                                  num_cores=NC, num_subcores=NS)
    f = pl.kernel(
        _sc_body,
        out_type=jax.ShapeDtypeStruct((NC * ZROWS, HKD), jnp.float32),
        mesh=mesh,
        scratch_types=[
            pltpu.VMEM_SHARED((ZPAD, HKD), jnp.float32),
            pltpu.VMEM((CPT, C), jnp.int32),
            pltpu.VMEM((CPT, C), jnp.int32),
            pltpu.VMEM((3, C, HKD), jnp.float32),
            pltpu.VMEM((3, C, HKD), jnp.float32),
        ] + [pltpu.SemaphoreType.DMA] * 9,
        compiler_params=pltpu.CompilerParams(use_tc_tiling_on_sc=False),
    )
    return f(hxa, hxb, we2, snd3, sidx3)


def kernel(dist, e_type, senders, receivers, W1, b1, W2, b2, Wh, bh, Wg, bg,
           Y_emb, X_emb):
    et2 = e_type.astype(jnp.int32).reshape(E, 1)
    rc2 = receivers.astype(jnp.int32).reshape(E, 1)
    snd3 = senders.astype(jnp.int32).reshape(NS, CPT, C)
    sidx3 = _make_sidx(et2, rc2).reshape(NS, CPT, C)

    elec = jnp.broadcast_to(X_emb[0:1, :], (N_ELEC, ED))
    for l in range(L):
        w1c = W1[l].transpose(1, 0, 2).reshape(DIST, 3 * DIST)
        b1c = b1[l].reshape(1, 3 * DIST)
        w2c = W2[l].reshape(3 * DIST, KD)
        b2c = b2[l]
        we2 = _edge_we(dist, et2, w1c, b1c, w2c, b2c)
        hxa, hxb = _embed_hx(elec, Wh[l], bh[l].reshape(1, KD), Y_emb)
        z = _sc_message(hxa, hxb, we2, snd3, sidx3)
        bg_row = (bg[l, 0] + bg[l, 1] + bg[l, 2]).reshape(1, ED)
        elec = _update(z, Wg[l], bg_row, elec)
    return elec


# speedup vs baseline: 4.8559x; 1.3220x over previous
import functools

import jax
import jax.numpy as jnp
from jax import lax
from jax.experimental import pallas as pl
from jax.experimental.pallas import tpu as pltpu
from jax.experimental.pallas import tpu_sc as plsc

N_NUC = 512
N_ELEC = 4096
N_NODES = N_NUC + N_ELEC
E = 73728
DIST = 16
KD = 128
ED = 64
L = 2

LN2 = 0.6931471805599453

NC = 2
NS = 16
NW = NC * NS
C = 32
EPW = E // NW
CPT = EPW // C
CB = C * KD * 4
NBUF = 3
ZROWS = 3 * N_ELEC
DUMP = ZROWS
ZPAD = ZROWS + 8
ROWS_PER_TILE = ZROWS // NS


def _ssp(x):
    return jnp.maximum(x, 0.0) + jnp.log1p(jnp.exp(-jnp.abs(x))) - LN2


def _edge_w_body(dist_ref, et_ref, w1_ref, b1_ref, w2_ref, b2_ref, we_ref):
    x = dist_ref[...]
    h = jnp.dot(x, w1_ref[...], preferred_element_type=jnp.float32)
    h = _ssp(h + b1_ref[...])
    et = et_ref[...]
    blk = x.shape[0]
    m_same = (et == 3).astype(jnp.float32)
    m_anti = (et == 4).astype(jnp.float32)
    m_nuc = (et == 1).astype(jnp.float32)
    mask48 = jnp.concatenate(
        [jnp.broadcast_to(m_same, (blk, DIST)),
         jnp.broadcast_to(m_anti, (blk, DIST)),
         jnp.broadcast_to(m_nuc, (blk, DIST))], axis=1)
    we = jnp.dot(h * mask48, w2_ref[...], preferred_element_type=jnp.float32)
    we = we + (m_same * b2_ref[0:1, :] + m_anti * b2_ref[1:2, :]
               + m_nuc * b2_ref[2:3, :])
    we_ref[...] = we


def _edge_we(dist, et2, w1c, b1c, w2c, b2c):
    blk = 2048
    grid = (E // blk,)
    return pl.pallas_call(
        _edge_w_body,
        grid=grid,
        in_specs=[
            pl.BlockSpec((blk, DIST), lambda i: (i, 0)),
            pl.BlockSpec((blk, 1), lambda i: (i, 0)),
            pl.BlockSpec((DIST, 3 * DIST), lambda i: (0, 0)),
            pl.BlockSpec((1, 3 * DIST), lambda i: (0, 0)),
            pl.BlockSpec((3 * DIST, KD), lambda i: (0, 0)),
            pl.BlockSpec((3, KD), lambda i: (0, 0)),
        ],
        out_specs=pl.BlockSpec((blk, KD), lambda i: (i, 0)),
        out_shape=jax.ShapeDtypeStruct((E, KD), jnp.float32),
    )(dist, et2, w1c, b1c, w2c, b2c)


def _sidx_body(et_ref, rc_ref, out_ref):
    et = et_ref[...]
    rc = rc_ref[...]
    bucket = jnp.where(et == 3, 0, jnp.where(et == 4, 1, jnp.where(et == 1, 2, 3)))
    live = (bucket < 3) & (rc >= N_NUC)
    out_ref[...] = jnp.where(live, bucket * N_ELEC + rc - N_NUC, DUMP).astype(jnp.int32)


def _make_sidx(et2, rc2):
    blk = 4096
    return pl.pallas_call(
        _sidx_body,
        grid=(E // blk,),
        in_specs=[pl.BlockSpec((blk, 1), lambda i: (i, 0)),
                  pl.BlockSpec((blk, 1), lambda i: (i, 0))],
        out_specs=pl.BlockSpec((blk, 1), lambda i: (i, 0)),
        out_shape=jax.ShapeDtypeStruct((E, 1), jnp.int32),
    )(et2, rc2)


def _embed_body(elec_ref, wh_ref, bh_ref, y_ref, out_ref):
    ex = jnp.dot(elec_ref[...], wh_ref[...],
                 preferred_element_type=jnp.float32) + bh_ref[...]
    out_ref[:N_NUC, :] = y_ref[...]
    out_ref[N_NUC:, :] = ex


def _embed_hx(elec, wh, bh_row, y_emb):
    return pl.pallas_call(
        _embed_body,
        out_shape=jax.ShapeDtypeStruct((N_NODES, KD), jnp.float32),
    )(elec, wh, bh_row, y_emb)


def _update_body(z_ref, wg_ref, bg_ref, elec_ref, out_ref):
    acc = elec_ref[...] + bg_ref[...]
    for ct in range(NC * 3):
        t = ct % 3
        zt = z_ref[ct * N_ELEC:(ct + 1) * N_ELEC, :]
        acc = acc + jnp.dot(zt, wg_ref[t], preferred_element_type=jnp.float32)
    out_ref[...] = acc


def _update(z, wg, bg_row, elec):
    return pl.pallas_call(
        _update_body,
        out_shape=jax.ShapeDtypeStruct((N_ELEC, ED), jnp.float32),
    )(z, wg, bg_row, elec)


def _sc_body(hx, we, ss4, zrs, out,
             z_sh, idx_b, rows_b, we_b,
             is0, is1, is2, gs0, gs1, gs2, ws0, ws1, ws2, ss0, ss1, ss2):
    c = lax.axis_index("c")
    s = lax.axis_index("s")
    wid = c * NS + s
    isem = (is0, is1, is2)
    gsem = (gs0, gs1, gs2)
    wsem = (ws0, ws1, ws2)
    ssem = (ss0, ss1, ss2)

    tbase = s * ROWS_PER_TILE
    pltpu.sync_copy(zrs.at[pl.ds(tbase, ROWS_PER_TILE), :],
                    z_sh.at[pl.ds(tbase, ROWS_PER_TILE), :])
    plsc.subcore_barrier()

    ebase = wid * EPW

    def _drain(sem, buf):
        pltpu.make_async_copy(we.at[pl.ds(0, C), :], buf, sem).wait()

    def _drain_i(b):
        pltpu.make_async_copy(ss4.at[wid, 0], idx_b.at[b], isem[b]).wait()

    def _issue_iw(k, b):
        pltpu.async_copy(ss4.at[wid, k], idx_b.at[b], isem[b])
        pltpu.async_copy(we.at[pl.ds(ebase + k * C, C), :],
                         we_b.at[pl.ds(b * C, C), :], wsem[b])

    def _issue_g(b):
        pltpu.async_copy(hx.at[idx_b.at[b, 0]], rows_b.at[pl.ds(b * C, C), :],
                         gsem[b])

    _issue_iw(0, 0)
    _issue_iw(1, 1)
    _drain_i(0)
    _issue_g(0)

    def _outer(ko, _):
        for j in range(NBUF):
            k = NBUF * ko + j
            b = j
            b1 = (j + 1) % NBUF
            b2 = (j + 2) % NBUF

            @pl.when(k + 1 < CPT)
            def _():
                _drain_i(b1)
                _issue_g(b1)

            _drain(wsem[b], we_b.at[pl.ds(b * C, C), :])
            _drain(gsem[b], rows_b.at[pl.ds(b * C, C), :])

            def _mrow(r, _):
                for q in range(KD // 16):
                    sl = pl.ds(q * 16, 16)
                    rows_b[b * C + r, sl] = rows_b[b * C + r, sl] * we_b[b * C + r, sl]
                return 0

            lax.fori_loop(0, C, _mrow, 0)
            pltpu.async_copy(rows_b.at[pl.ds(b * C, C), :],
                             z_sh.at[idx_b.at[b, 1]], ssem[b], add=True)

            @pl.when(k >= 1)
            def _():
                _drain(ssem[b2], rows_b.at[pl.ds(b2 * C, C), :])

            @pl.when(k + 2 < CPT)
            def _():
                _issue_iw(k + 2, b2)
        return 0

    lax.fori_loop(0, CPT // NBUF, _outer, 0)
    _drain(ssem[(CPT - 1) % NBUF], rows_b.at[pl.ds(((CPT - 1) % NBUF) * C, C), :])
    plsc.subcore_barrier()

    pltpu.sync_copy(z_sh.at[pl.ds(tbase, ROWS_PER_TILE), :],
                    out.at[pl.ds(c * ZROWS + tbase, ROWS_PER_TILE), :])


def _sc_message(hx, we, ss4, zrs):
    mesh = plsc.VectorSubcoreMesh(core_axis_name="c", subcore_axis_name="s",
                                  num_cores=NC, num_subcores=NS)
    f = pl.kernel(
        _sc_body,
        out_type=jax.ShapeDtypeStruct((NC * ZROWS, KD), jnp.float32),
        mesh=mesh,
        scratch_types=[
            pltpu.VMEM_SHARED((ZPAD, KD), jnp.float32),
            pltpu.VMEM((NBUF, 2, C), jnp.int32),
            pltpu.VMEM((NBUF * C, KD), jnp.float32),
            pltpu.VMEM((NBUF * C, KD), jnp.float32),
        ] + [pltpu.SemaphoreType.DMA] * 12,
    )
    return f(hx, we, ss4, zrs)


def kernel(dist, e_type, senders, receivers, W1, b1, W2, b2, Wh, bh, Wg, bg,
           Y_emb, X_emb):
    et2 = e_type.astype(jnp.int32).reshape(E, 1)
    rc2 = receivers.astype(jnp.int32).reshape(E, 1)
    snd4 = senders.astype(jnp.int32).reshape(NW, CPT, C)
    sidx4 = _make_sidx(et2, rc2).reshape(NW, CPT, C)
    ss4 = jnp.stack([snd4, sidx4], axis=2)
    zrs = jnp.zeros((ZROWS, KD), jnp.float32)

    elec = jnp.broadcast_to(X_emb[0:1, :], (N_ELEC, ED))
    for l in range(L):
        w1c = W1[l].transpose(1, 0, 2).reshape(DIST, 3 * DIST)
        b1c = b1[l].reshape(1, 3 * DIST)
        w2c = W2[l].reshape(3 * DIST, KD)
        b2c = b2[l]
        we = _edge_we(dist, et2, w1c, b1c, w2c, b2c)
        hx = _embed_hx(elec, Wh[l], bh[l].reshape(1, KD), Y_emb)
        z = _sc_message(hx, we, ss4, zrs)
        bg_row = (bg[l, 0] + bg[l, 1] + bg[l, 2]).reshape(1, ED)
        elec = _update(z, Wg[l], bg_row, elec)
    return elec
